# baseline (speedup 1.0000x reference)
import functools

import jax
import jax.numpy as jnp
from jax import lax
from jax.experimental import pallas as pl
from jax.experimental.pallas import tpu as pltpu
from jax.experimental.pallas import tpu_sc as plsc

_GAMMA = 12.0
_D = 64
_B = 16384
_NC = 2
_NS = 16
_L = 16
_NW = _NC * _NS
_BPW = _B // _NW
_CH = 128
_NCH = _BPW // _CH

_mesh = plsc.VectorSubcoreMesh(core_axis_name="c", subcore_axis_name="s")


@functools.partial(
    pl.kernel,
    out_type=jax.ShapeDtypeStruct((_NW, _BPW), jnp.float32),
    mesh=_mesh,
    scratch_types=[
        pltpu.VMEM((_NCH, _CH), jnp.int32),
        pltpu.VMEM((_NCH, _CH), jnp.int32),
        pltpu.VMEM((_NCH, _CH), jnp.int32),
        pltpu.VMEM((_BPW, _D), jnp.float32),
        pltpu.VMEM((_BPW, _D), jnp.float32),
        pltpu.VMEM((_BPW, _D), jnp.float32),
        pltpu.VMEM((_BPW,), jnp.float32),
        pltpu.SemaphoreType.DMA,
    ],
    compiler_params=pltpu.CompilerParams(
        needs_layout_passes=False, use_tc_tiling_on_sc=False),
)
def _kge_score(ent_hbm, relemb_hbm, head_hbm, rel_hbm, tail_hbm, out_hbm,
               idx_h, idx_r, idx_t, h_v, r_v, t_v, o_v, sem):
    wid = lax.axis_index("s") * _NC + lax.axis_index("c")

    pltpu.sync_copy(head_hbm.at[wid], idx_h)
    pltpu.sync_copy(rel_hbm.at[wid], idx_r)
    pltpu.sync_copy(tail_hbm.at[wid], idx_t)

    copies = []
    for j in range(_NCH):
        sl = pl.ds(j * _CH, _CH)
        copies.append(pltpu.async_copy(ent_hbm.at[idx_h.at[j]], h_v.at[sl], sem))
        copies.append(pltpu.async_copy(relemb_hbm.at[idx_r.at[j]], r_v.at[sl], sem))
        copies.append(pltpu.async_copy(ent_hbm.at[idx_t.at[j]], t_v.at[sl], sem))
    for c in copies:
        c.wait()

    def group(g, carry):
        rows = g * _L + lax.iota(jnp.int32, _L)
        acc = jnp.zeros((_L,), jnp.float32)
        for d in range(_D):
            col = jnp.full((_L,), d, jnp.int32)
            hv = plsc.load_gather(h_v, [rows, col])
            rv = plsc.load_gather(r_v, [rows, col])
            tv = plsc.load_gather(t_v, [rows, col])
            diff = hv + rv - tv
            acc = acc + diff * diff
        x = acc + 1e-12
        seed = plsc.bitcast(
            (plsc.bitcast(x, jnp.int32) >> 1) + 0x1FBD1DF5, jnp.float32)
        y = 0.5 * (seed + x / seed)
        y = 0.5 * (y + x / y)
        o_v[pl.ds(g * _L, _L)] = _GAMMA - y
        return carry

    lax.fori_loop(0, _BPW // _L, group, 0)
    pltpu.sync_copy(o_v, out_hbm.at[wid])


def kernel(entity_emb, relation_emb, head, rel, tail):
    head3 = head.reshape(_NW, _NCH, _CH)
    rel3 = rel.reshape(_NW, _NCH, _CH)
    tail3 = tail.reshape(_NW, _NCH, _CH)
    out = _kge_score(entity_emb, relation_emb, head3, rel3, tail3)
    return out.reshape(_B)


# speedup vs baseline: 1.4322x; 1.4322x over previous
import functools

import jax
import jax.numpy as jnp
from jax import lax
from jax.experimental import pallas as pl
from jax.experimental.pallas import tpu as pltpu
from jax.experimental.pallas import tpu_sc as plsc

_GAMMA = 12.0
_D = 64
_B = 16384
_NC = 2
_NS = 16
_L = 16
_NW = _NC * _NS
_BPW = _B // _NW
_CH = 128
_NCH = _BPW // _CH

_mesh = plsc.VectorSubcoreMesh(core_axis_name="c", subcore_axis_name="s")


@functools.partial(
    pl.kernel,
    out_type=jax.ShapeDtypeStruct((_NW, _BPW), jnp.float32),
    mesh=_mesh,
    scratch_types=[
        pltpu.VMEM((_NCH, _CH), jnp.int32),
        pltpu.VMEM((_NCH, _CH), jnp.int32),
        pltpu.VMEM((_NCH, _CH), jnp.int32),
        pltpu.VMEM((_BPW, _D), jnp.float32),
        pltpu.VMEM((_BPW, _D), jnp.float32),
        pltpu.VMEM((_BPW, _D), jnp.float32),
        pltpu.VMEM((_BPW,), jnp.float32),
        pltpu.SemaphoreType.DMA,
    ],
    compiler_params=pltpu.CompilerParams(
        needs_layout_passes=False, use_tc_tiling_on_sc=False),
)
def _kge_score(ent_hbm, relemb_hbm, head_hbm, rel_hbm, tail_hbm, out_hbm,
               idx_h, idx_r, idx_t, h_v, r_v, t_v, o_v, sem):
    wid = lax.axis_index("s") * _NC + lax.axis_index("c")

    pltpu.sync_copy(head_hbm.at[wid], idx_h)
    pltpu.sync_copy(rel_hbm.at[wid], idx_r)
    pltpu.sync_copy(tail_hbm.at[wid], idx_t)

    copies = []
    for j in range(_NCH):
        sl = pl.ds(j * _CH, _CH)
        copies.append(pltpu.async_copy(ent_hbm.at[idx_h.at[j]], h_v.at[sl], sem))
        copies.append(pltpu.async_copy(relemb_hbm.at[idx_r.at[j]], r_v.at[sl], sem))
        copies.append(pltpu.async_copy(ent_hbm.at[idx_t.at[j]], t_v.at[sl], sem))
    for c in copies:
        c.wait()

    lanes = lax.iota(jnp.int32, _L)

    def group(g, carry):
        acc = jnp.zeros((_L,), jnp.float32)
        for row in range(_L):
            i = g * _L + row
            s = jnp.zeros((_L,), jnp.float32)
            for c in range(_D // _L):
                sl = pl.ds(c * _L, _L)
                diff = h_v[i, sl] + r_v[i, sl] - t_v[i, sl]
                s = s + diff * diff
            tot = lax.reduce_sum_p.bind(s, axes=(0,))
            acc = jnp.where(lanes == row, tot, acc)
        x = acc + 1e-12
        seed = plsc.bitcast(
            (plsc.bitcast(x, jnp.int32) >> 1) + 0x1FBD1DF5, jnp.float32)
        y = 0.5 * (seed + x / seed)
        y = 0.5 * (y + x / y)
        o_v[pl.ds(g * _L, _L)] = _GAMMA - y
        return carry

    lax.fori_loop(0, _BPW // _L, group, 0)
    pltpu.sync_copy(o_v, out_hbm.at[wid])


def kernel(entity_emb, relation_emb, head, rel, tail):
    head3 = head.reshape(_NW, _NCH, _CH)
    rel3 = rel.reshape(_NW, _NCH, _CH)
    tail3 = tail.reshape(_NW, _NCH, _CH)
    out = _kge_score(entity_emb, relation_emb, head3, rel3, tail3)
    return out.reshape(_B)
